# baseline (speedup 1.0000x reference)
import functools

import jax
import jax.numpy as jnp
from jax import lax
from jax.experimental import pallas as pl
from jax.experimental.pallas import tpu as pltpu
from jax.experimental.pallas import tpu_sc as plsc

B, N, D, H, E = 32, 512, 16, 128, 64
S = B * N
NC, NS, L = 2, 16, 16
NW = NC * NS
SLOTS_W = S // NW
G = 8
ROWS = G * D
NG = SLOTS_W // G
HS = H // L


def _lane_bcast(vec, d):
    return lax.gather(
        vec, jnp.broadcast_to(d, (L,))[:, None],
        lax.GatherDimensionNumbers(
            offset_dims=(), collapsed_slice_dims=(0,), start_index_map=(0,)),
        slice_sizes=(1,),
        mode=lax.GatherScatterMode.PROMISE_IN_BOUNDS)


def _make_agg_kernel():
    mesh = plsc.VectorSubcoreMesh(core_axis_name="c", subcore_axis_name="s")

    @functools.partial(
        pl.kernel,
        out_type=[jax.ShapeDtypeStruct((S, H), jnp.float32),
                  jax.ShapeDtypeStruct((S, H), jnp.float32)],
        name="neighbor_agg",
        mesh=mesh,
        scratch_types=[
            pltpu.VMEM((2, SLOTS_W * D), jnp.int32),
            pltpu.VMEM((2, SLOTS_W * D), jnp.float32),
            pltpu.VMEM((N, H), jnp.float32),
            pltpu.VMEM((2, G, H), jnp.float32),
            pltpu.SemaphoreType.DMA,
            pltpu.SemaphoreType.DMA,
        ],
    )
    def agg_kernel(nh, idx_in, w_in, idx_out, w_out, agg_in, agg_out,
                   idx_v, w_v, table_v, out_v, so0, so1):
        wid = lax.axis_index("s") * NC + lax.axis_index("c")
        base = wid * SLOTS_W
        sos = (so0, so1)
        idx_hbm = (idx_in, idx_out)
        w_hbm = (w_in, w_out)
        aggs = (agg_in, agg_out)

        pltpu.sync_copy(nh.at[pl.ds(base, N)], table_v)
        for dirn in range(2):
            pltpu.sync_copy(idx_hbm[dirn].at[pl.ds(base * D, SLOTS_W * D)],
                            idx_v.at[dirn])
            pltpu.sync_copy(w_hbm[dirn].at[pl.ds(base * D, SLOTS_W * D)],
                            w_v.at[dirn])

        for dirn in range(2):
            def consume(g, b, dirn=dirn):
                @pl.when(g >= 2)
                def _():
                    pltpu.make_async_copy(
                        out_v.at[b], aggs[dirn].at[pl.ds(base, G)],
                        sos[b]).wait()
                @plsc.parallel_loop(0, G, unroll=2)
                def _slot(s, b=b, g=g, dirn=dirn):
                    off = g * ROWS + s * D
                    iv = idx_v[dirn, pl.ds(off, L)]
                    wv16 = w_v[dirn, pl.ds(off, L)]
                    accs = [jnp.zeros((L,), jnp.float32) for _ in range(HS)]
                    for d in range(D):
                        r = iv[d]
                        wb = _lane_bcast(wv16, d)
                        for h in range(HS):
                            accs[h] = (accs[h]
                                       + wb * table_v[r, pl.ds(h * L, L)])
                    for h in range(HS):
                        out_v[b, s, pl.ds(h * L, L)] = accs[h]
                pltpu.async_copy(
                    out_v.at[b], aggs[dirn].at[pl.ds(base + g * G, G)],
                    sos[b])

            @pl.loop(0, NG // 2)
            def _pipe(i, dirn=dirn):
                t0 = 2 * i
                consume(t0, 0)
                consume(t0 + 1, 1)

            for b in range(2):
                pltpu.make_async_copy(
                    out_v.at[b], aggs[dirn].at[pl.ds(base, G)],
                    sos[b]).wait()

    return agg_kernel


_agg = _make_agg_kernel()

BLK = 1024
NBLK = S // BLK


def _hist_body(lab_ref, w_ref, hin_ref, hout_ref):
    lin = lab_ref[0:D, :]
    win = w_ref[0:D, :]
    lout = lab_ref[D:2 * D, :]
    wout = w_ref[D:2 * D, :]
    for e in range(E):
        hin_ref[e, :] = jnp.sum(jnp.where(lin == e, win, 0.0), axis=0)
        hout_ref[e, :] = jnp.sum(jnp.where(lout == e, wout, 0.0), axis=0)


_hist = pl.pallas_call(
    _hist_body,
    grid=(NBLK,),
    in_specs=[pl.BlockSpec((2 * D, BLK), lambda i: (0, i)),
              pl.BlockSpec((2 * D, BLK), lambda i: (0, i))],
    out_specs=[pl.BlockSpec((E, BLK), lambda i: (0, i)),
               pl.BlockSpec((E, BLK), lambda i: (0, i))],
    out_shape=[jax.ShapeDtypeStruct((E, S), jnp.float32),
               jax.ShapeDtypeStruct((E, S), jnp.float32)],
)


def _gates_body(inagg_ref, outagg_ref, hin_ref, hout_ref, cell_ref,
                wni_ref, uti_ref, wno_ref, uto_ref, edge_ref, b_ref,
                out_ref):
    bf = jnp.bfloat16
    eui = jnp.dot(edge_ref[...], uti_ref[...], preferred_element_type=jnp.float32)
    euo = jnp.dot(edge_ref[...], uto_ref[...], preferred_element_type=jnp.float32)
    dn = (((0,), (0,)), ((), ()))
    pre = (jnp.dot(inagg_ref[...].astype(bf), wni_ref[...].astype(bf),
                   preferred_element_type=jnp.float32)
           + lax.dot_general(hin_ref[...].astype(bf), eui.astype(bf), dn,
                             preferred_element_type=jnp.float32)
           + jnp.dot(outagg_ref[...].astype(bf), wno_ref[...].astype(bf),
                     preferred_element_type=jnp.float32)
           + lax.dot_general(hout_ref[...].astype(bf), euo.astype(bf), dn,
                             preferred_element_type=jnp.float32)
           + b_ref[...])
    i_g = jax.nn.sigmoid(pre[:, 0:H])
    o_g = jax.nn.sigmoid(pre[:, H:2 * H])
    f_g = jax.nn.sigmoid(pre[:, 2 * H:3 * H])
    c_t = jnp.tanh(pre[:, 3 * H:4 * H])
    new_cell = f_g * cell_ref[...] + i_g * c_t
    out_ref[...] = o_g * jnp.tanh(new_cell)


def _row_spec(cols):
    return pl.BlockSpec((BLK, cols), lambda i: (i, 0))


def _full_spec(shape):
    return pl.BlockSpec(shape, lambda i: (0,) * len(shape))


_gates = pl.pallas_call(
    _gates_body,
    grid=(NBLK,),
    in_specs=[
        _row_spec(H), _row_spec(H),
        pl.BlockSpec((E, BLK), lambda i: (0, i)),
        pl.BlockSpec((E, BLK), lambda i: (0, i)),
        _row_spec(H),
        _full_spec((H, 4 * H)), _full_spec((H, 4 * H)),
        _full_spec((H, 4 * H)), _full_spec((H, 4 * H)),
        _full_spec((E, H)), _full_spec((1, 4 * H)),
    ],
    out_specs=pl.BlockSpec((BLK, H), lambda i: (i, 0)),
    out_shape=jax.ShapeDtypeStruct((S, H), jnp.float32),
)


def kernel(node_hidden, cell, in_node_mask, out_node_mask, W_in, U_in,
           W_out, U_out, b, edge_emb, in_nodes, in_labels, out_nodes,
           out_labels):
    nh = node_hidden.reshape(S, H)
    idx_in = in_nodes.reshape(S * D)
    idx_out = out_nodes.reshape(S * D)
    w_in = in_node_mask.reshape(S * D)
    w_out = out_node_mask.reshape(S * D)
    lab_t = jnp.concatenate([in_labels.reshape(S, D),
                             out_labels.reshape(S, D)], axis=1).T
    w_t = jnp.concatenate([in_node_mask.reshape(S, D),
                           out_node_mask.reshape(S, D)], axis=1).T
    hin, hout = _hist(lab_t, w_t)
    agg_in, agg_out = _agg(nh, idx_in, w_in, idx_out, w_out)

    wni = W_in.transpose(1, 0, 2).reshape(H, 4 * H)
    uti = U_in.transpose(1, 0, 2).reshape(H, 4 * H)
    wno = W_out.transpose(1, 0, 2).reshape(H, 4 * H)
    uto = U_out.transpose(1, 0, 2).reshape(H, 4 * H)
    b_flat = b.reshape(1, 4 * H)

    new_h = _gates(agg_in, agg_out, hin, hout, cell.reshape(S, H),
                   wni, uti, wno, uto, edge_emb, b_flat)
    return new_h.reshape(B, N, H)


# speedup vs baseline: 1.6890x; 1.6890x over previous
import functools

import jax
import jax.numpy as jnp
from jax import lax
from jax.experimental import pallas as pl
from jax.experimental.pallas import tpu as pltpu
from jax.experimental.pallas import tpu_sc as plsc

B, N, D, H, E = 32, 512, 16, 128, 64
S = B * N
NC, NS, L = 2, 16, 16
NW = NC * NS
SLOTS_W = S // NW
G = 8
ROWS = G * D
NG = SLOTS_W // G
HS = H // L


def _lane_bcast(vec, d):
    return lax.gather(
        vec, jnp.broadcast_to(d, (L,))[:, None],
        lax.GatherDimensionNumbers(
            offset_dims=(), collapsed_slice_dims=(0,), start_index_map=(0,)),
        slice_sizes=(1,),
        mode=lax.GatherScatterMode.PROMISE_IN_BOUNDS)


def _make_agg_kernel():
    mesh = plsc.VectorSubcoreMesh(core_axis_name="c", subcore_axis_name="s")

    @functools.partial(
        pl.kernel,
        out_type=[jax.ShapeDtypeStruct((S, H), jnp.float32),
                  jax.ShapeDtypeStruct((S, H), jnp.float32)],
        name="neighbor_agg",
        mesh=mesh,
        scratch_types=[
            pltpu.VMEM((2, SLOTS_W * D), jnp.int32),
            pltpu.VMEM((2, SLOTS_W * D), jnp.float32),
            pltpu.VMEM((N, H), jnp.float32),
            pltpu.VMEM((2, G, H), jnp.float32),
            pltpu.SemaphoreType.DMA,
            pltpu.SemaphoreType.DMA,
        ],
    )
    def agg_kernel(nh, idx_in, w_in, idx_out, w_out, agg_in, agg_out,
                   idx_v, w_v, table_v, out_v, so0, so1):
        wid = lax.axis_index("s") * NC + lax.axis_index("c")
        base = wid * SLOTS_W
        sos = (so0, so1)
        idx_hbm = (idx_in, idx_out)
        w_hbm = (w_in, w_out)
        aggs = (agg_in, agg_out)

        pltpu.sync_copy(nh.at[pl.ds(base, N)], table_v)
        for dirn in range(2):
            pltpu.sync_copy(idx_hbm[dirn].at[pl.ds(base * D, SLOTS_W * D)],
                            idx_v.at[dirn])
            pltpu.sync_copy(w_hbm[dirn].at[pl.ds(base * D, SLOTS_W * D)],
                            w_v.at[dirn])

        for dirn in range(2):
            def consume(g, b, dirn=dirn):
                @pl.when(g >= 2)
                def _():
                    pltpu.make_async_copy(
                        out_v.at[b], aggs[dirn].at[pl.ds(base, G)],
                        sos[b]).wait()
                @pl.loop(0, G, unroll=2)
                def _slot(s, b=b, g=g, dirn=dirn):
                    off = g * ROWS + s * D
                    iv = idx_v[dirn, pl.ds(off, L)]
                    wv16 = w_v[dirn, pl.ds(off, L)]
                    accs = [jnp.zeros((L,), jnp.float32) for _ in range(HS)]
                    for d in range(D):
                        r = iv[d]
                        wb = _lane_bcast(wv16, d)
                        for h in range(HS):
                            accs[h] = (accs[h]
                                       + wb * table_v[r, pl.ds(h * L, L)])
                    for h in range(HS):
                        out_v[b, s, pl.ds(h * L, L)] = accs[h]
                pltpu.async_copy(
                    out_v.at[b], aggs[dirn].at[pl.ds(base + g * G, G)],
                    sos[b])

            @pl.loop(0, NG // 2)
            def _pipe(i, dirn=dirn):
                t0 = 2 * i
                consume(t0, 0)
                consume(t0 + 1, 1)

            for b in range(2):
                pltpu.make_async_copy(
                    out_v.at[b], aggs[dirn].at[pl.ds(base, G)],
                    sos[b]).wait()

    return agg_kernel


_agg = _make_agg_kernel()

BLK = 1024
NBLK = S // BLK


def _hist_body(lab_ref, w_ref, hin_ref, hout_ref):
    lin = lab_ref[0:D, :]
    win = w_ref[0:D, :]
    lout = lab_ref[D:2 * D, :]
    wout = w_ref[D:2 * D, :]
    for e in range(E):
        hin_ref[e, :] = jnp.sum(jnp.where(lin == e, win, 0.0), axis=0)
        hout_ref[e, :] = jnp.sum(jnp.where(lout == e, wout, 0.0), axis=0)


_hist = pl.pallas_call(
    _hist_body,
    grid=(NBLK,),
    in_specs=[pl.BlockSpec((2 * D, BLK), lambda i: (0, i)),
              pl.BlockSpec((2 * D, BLK), lambda i: (0, i))],
    out_specs=[pl.BlockSpec((E, BLK), lambda i: (0, i)),
               pl.BlockSpec((E, BLK), lambda i: (0, i))],
    out_shape=[jax.ShapeDtypeStruct((E, S), jnp.float32),
               jax.ShapeDtypeStruct((E, S), jnp.float32)],
)


def _gates_body(inagg_ref, outagg_ref, hin_ref, hout_ref, cell_ref,
                wni_ref, uti_ref, wno_ref, uto_ref, edge_ref, b_ref,
                out_ref):
    eui = jnp.dot(edge_ref[...], uti_ref[...], preferred_element_type=jnp.float32)
    euo = jnp.dot(edge_ref[...], uto_ref[...], preferred_element_type=jnp.float32)
    dn = (((0,), (0,)), ((), ()))
    pre = (jnp.dot(inagg_ref[...], wni_ref[...], preferred_element_type=jnp.float32)
           + lax.dot_general(hin_ref[...], eui, dn, preferred_element_type=jnp.float32)
           + jnp.dot(outagg_ref[...], wno_ref[...], preferred_element_type=jnp.float32)
           + lax.dot_general(hout_ref[...], euo, dn, preferred_element_type=jnp.float32)
           + b_ref[...])
    i_g = jax.nn.sigmoid(pre[:, 0:H])
    o_g = jax.nn.sigmoid(pre[:, H:2 * H])
    f_g = jax.nn.sigmoid(pre[:, 2 * H:3 * H])
    c_t = jnp.tanh(pre[:, 3 * H:4 * H])
    new_cell = f_g * cell_ref[...] + i_g * c_t
    out_ref[...] = o_g * jnp.tanh(new_cell)


def _row_spec(cols):
    return pl.BlockSpec((BLK, cols), lambda i: (i, 0))


def _full_spec(shape):
    return pl.BlockSpec(shape, lambda i: (0,) * len(shape))


_gates = pl.pallas_call(
    _gates_body,
    grid=(NBLK,),
    in_specs=[
        _row_spec(H), _row_spec(H),
        pl.BlockSpec((E, BLK), lambda i: (0, i)),
        pl.BlockSpec((E, BLK), lambda i: (0, i)),
        _row_spec(H),
        _full_spec((H, 4 * H)), _full_spec((H, 4 * H)),
        _full_spec((H, 4 * H)), _full_spec((H, 4 * H)),
        _full_spec((E, H)), _full_spec((1, 4 * H)),
    ],
    out_specs=pl.BlockSpec((BLK, H), lambda i: (i, 0)),
    out_shape=jax.ShapeDtypeStruct((S, H), jnp.float32),
)


def kernel(node_hidden, cell, in_node_mask, out_node_mask, W_in, U_in,
           W_out, U_out, b, edge_emb, in_nodes, in_labels, out_nodes,
           out_labels):
    nh = node_hidden.reshape(S, H)
    idx_in = in_nodes.reshape(S * D)
    idx_out = out_nodes.reshape(S * D)
    w_in = in_node_mask.reshape(S * D)
    w_out = out_node_mask.reshape(S * D)
    lab_t = jnp.concatenate([in_labels.reshape(S, D),
                             out_labels.reshape(S, D)], axis=1).T
    w_t = jnp.concatenate([in_node_mask.reshape(S, D),
                           out_node_mask.reshape(S, D)], axis=1).T
    hin, hout = _hist(lab_t, w_t)
    agg_in, agg_out = _agg(nh, idx_in, w_in, idx_out, w_out)

    wni = W_in.transpose(1, 0, 2).reshape(H, 4 * H)
    uti = U_in.transpose(1, 0, 2).reshape(H, 4 * H)
    wno = W_out.transpose(1, 0, 2).reshape(H, 4 * H)
    uto = U_out.transpose(1, 0, 2).reshape(H, 4 * H)
    b_flat = b.reshape(1, 4 * H)

    new_h = _gates(agg_in, agg_out, hin, hout, cell.reshape(S, H),
                   wni, uti, wno, uto, edge_emb, b_flat)
    return new_h.reshape(B, N, H)
